# baseline (speedup 1.0000x reference)
import functools

import jax
import jax.numpy as jnp
from jax.experimental import pallas as pl
from jax.experimental.pallas import tpu as pltpu

_BLK = 1024


def _router_kernel(x_ref, w_ref, o_ref):
    o_ref[...] = jax.lax.dot_general(
        x_ref[...], w_ref[...],
        dimension_numbers=(((1,), (1,)), ((), ())),
        preferred_element_type=jnp.float32,
        precision=jax.lax.Precision.HIGHEST,
    )


@functools.partial(jax.jit, static_argnames=())
def kernel(x, weight):
    hidden = weight.shape[1]
    xf = x.reshape(-1, hidden)
    rows = xf.shape[0]
    n_exp = weight.shape[0]
    grid = rows // _BLK
    out = pl.pallas_call(
        _router_kernel,
        grid=(grid,),
        in_specs=[
            pl.BlockSpec((_BLK, hidden), lambda i: (i, 0)),
            pl.BlockSpec((n_exp, hidden), lambda i: (0, 0)),
        ],
        out_specs=pl.BlockSpec((_BLK, n_exp), lambda i: (i, 0)),
        out_shape=jax.ShapeDtypeStruct((rows, n_exp), jnp.float32),
    )(xf, weight)
    return out


# speedup vs baseline: 2.4296x; 2.4296x over previous
import functools

import jax
import jax.numpy as jnp
from jax.experimental import pallas as pl
from jax.experimental.pallas import tpu as pltpu

_BLK = 1024


def _router_kernel(x_ref, w_ref, o_ref):
    o_ref[...] = jax.lax.dot_general(
        x_ref[...], w_ref[...],
        dimension_numbers=(((1,), (1,)), ((), ())),
        preferred_element_type=jnp.float32,
        precision=jax.lax.Precision.DEFAULT,
    )


@functools.partial(jax.jit, static_argnames=())
def kernel(x, weight):
    hidden = weight.shape[1]
    xf = x.reshape(-1, hidden)
    rows = xf.shape[0]
    n_exp = weight.shape[0]
    grid = rows // _BLK
    out = pl.pallas_call(
        _router_kernel,
        grid=(grid,),
        in_specs=[
            pl.BlockSpec((_BLK, hidden), lambda i: (i, 0)),
            pl.BlockSpec((n_exp, hidden), lambda i: (0, 0)),
        ],
        out_specs=pl.BlockSpec((_BLK, n_exp), lambda i: (i, 0)),
        out_shape=jax.ShapeDtypeStruct((rows, n_exp), jnp.float32),
    )(xf, weight)
    return out
